# baseline (speedup 1.0000x reference)
import functools

import jax
import jax.numpy as jnp
from jax import lax
from jax.experimental import pallas as pl
from jax.experimental.pallas import tpu as pltpu
from jax.experimental.pallas import tpu_sc as plsc

LN_EPS = 1e-12

_NC = 2
_NS = 16
_NW = _NC * _NS

_GROUP_SEQS = (64, 192, 320, 448)
_CHUNK = 128
_SEQS_PER_BLOCK = 16


def _sc_gather(ids3, table):
  nw, n_ch, chunk = ids3.shape
  e = table.shape[1]
  n = nw * n_ch * chunk
  rows_per_w = n_ch * chunk
  mesh = plsc.VectorSubcoreMesh(core_axis_name="c", subcore_axis_name="s")

  assert n_ch >= 2 and n_ch % 2 == 0

  @functools.partial(
      pl.kernel,
      mesh=mesh,
      out_type=jax.ShapeDtypeStruct((n // 2, e), jnp.int32),
      scratch_types=[
          pltpu.VMEM((n_ch, chunk), jnp.int32),
          pltpu.VMEM((chunk, e), jnp.float32),
          pltpu.VMEM((chunk, e), jnp.float32),
          pltpu.VMEM((chunk // 2, e), jnp.int32),
          pltpu.VMEM((chunk // 2, e), jnp.int32),
          pltpu.SemaphoreType.DMA,
          pltpu.SemaphoreType.DMA,
      ],
  )
  def gather_kernel(ids_hbm, tab_hbm, out_hbm, idx_v, rva, rvb, oba, obb,
                    sema, semb):
    wid = lax.axis_index("s") * _NC + lax.axis_index("c")
    base = wid * (rows_per_w // 2)
    pltpu.sync_copy(ids_hbm.at[wid], idx_v)
    bufs = ((rva, oba, sema), (rvb, obb, semb))

    def gstart(i, b):
      rv, _, sem = bufs[b]
      pltpu.async_copy(tab_hbm.at[idx_v.at[i]], rv, sem)

    def gwait(i, b):
      rv, _, sem = bufs[b]
      pltpu.make_async_copy(tab_hbm.at[idx_v.at[i]], rv, sem).wait()

    def to_bf16(b):
      rv, ob, _ = bufs[b]

      def rbody(r, carry):
        for gcol in range(e // 16):
          sl = pl.ds(16 * gcol, 16)
          lo = lax.bitcast_convert_type(rv[2 * r, sl], jnp.int32)
          hi = lax.bitcast_convert_type(rv[2 * r + 1, sl], jnp.int32)
          lo16 = ((lo + 0x7FFF + ((lo >> 16) & 1)) >> 16) & 0xFFFF
          hi16 = ((hi + 0x7FFF + ((hi >> 16) & 1)) >> 16) & 0xFFFF
          ob[r, sl] = lo16 | (hi16 << 16)
        return carry

      lax.fori_loop(0, chunk // 2, rbody, 0)

    def drain(i, b):
      _, ob, _ = bufs[b]
      pltpu.sync_copy(
          ob, out_hbm.at[pl.ds(base + i * (chunk // 2), chunk // 2)]
      )

    gstart(0, 0)
    gstart(1, 1)

    def body(k, carry):
      for b in range(2):
        i = 2 * k + b
        gwait(i, b)
        to_bf16(b)
        drain(i, b)
        gstart(i + 2, b)
      return carry

    lax.fori_loop(0, n_ch // 2 - 1, body, 0)
    for b in range(2):
      i = n_ch - 2 + b
      gwait(i, b)
      to_bf16(b)
      drain(i, b)

  return gather_kernel(ids3, table)


def _tc_fused_group(x, rows, pos_tiled, g, b, proj, bias, blk0, n_total,
                    out_buf):
  e = x.shape[1]
  r = pos_tiled.shape[0]
  h = proj.shape[1]
  grid_n = rows // r
  row0 = blk0

  def body(*refs):
    x_ref, p_ref, g_ref, b_ref, k_ref, bias_ref = refs[:6]
    o_ref = refs[-1]
    xw = pltpu.bitcast(x_ref[...], jnp.bfloat16)
    xv = xw.astype(jnp.float32) + p_ref[...]
    mu = jnp.mean(xv, axis=1, keepdims=True)
    xc = xv - mu
    var = jnp.mean(xc * xc, axis=1, keepdims=True)
    y = xc * lax.rsqrt(var + LN_EPS) * g_ref[...] + b_ref[...]
    o_ref[...] = (
        jnp.dot(y, k_ref[...], preferred_element_type=jnp.float32)
        + bias_ref[...]
    )

  in_specs = [
      pl.BlockSpec((r // 2, e), lambda i: (i, 0)),
      pl.BlockSpec((r, e), lambda i: (0, 0)),
      pl.BlockSpec((1, e), lambda i: (0, 0)),
      pl.BlockSpec((1, e), lambda i: (0, 0)),
      pl.BlockSpec((e, h), lambda i: (0, 0)),
      pl.BlockSpec((1, h), lambda i: (0, 0)),
  ]
  args = [x, pos_tiled, g, b, proj, bias]
  aliases = {}
  if out_buf is not None:
    in_specs.append(pl.BlockSpec(memory_space=pl.ANY))
    args.append(out_buf)
    aliases = {6: 0}

  return pl.pallas_call(
      body,
      grid=(grid_n,),
      in_specs=in_specs,
      out_specs=pl.BlockSpec((r, h), lambda i: (row0 + i, 0)),
      out_shape=jax.ShapeDtypeStruct((n_total, h), jnp.float32),
      input_output_aliases=aliases,
  )(*args)


def kernel(input_ids, word_emb, pos_emb, ln_scale, ln_bias, kernel, bias):
  bsz, seq = input_ids.shape
  n = bsz * seq
  ids_flat = input_ids.astype(jnp.int32).reshape(-1)

  bounds = []
  row = 0
  for gs in _GROUP_SEQS:
    bounds.append((row, gs * seq))
    row += gs * seq

  unit = 2 * _NW * _CHUNK
  gathered = []
  for r0, rows in bounds:
    rows_pad = -(-rows // unit) * unit
    ids_g = lax.dynamic_slice(ids_flat, (r0,), (rows,))
    if rows_pad != rows:
      ids_g = jnp.concatenate([ids_g, ids_g[: rows_pad - rows]])
    gathered.append(
        _sc_gather(
            ids_g.reshape(_NW, rows_pad // (_NW * _CHUNK), _CHUNK), word_emb
        )
    )

  r_block = _SEQS_PER_BLOCK * seq
  pos_tiled = jnp.tile(pos_emb[:seq], (_SEQS_PER_BLOCK, 1))
  g2 = ln_scale[None, :]
  b2 = ln_bias[None, :]
  bias2 = bias[None, :]
  out = None
  for i, (r0, rows) in enumerate(bounds):
    out = _tc_fused_group(
        gathered[i], rows, pos_tiled, g2, b2, kernel, bias2, r0 // r_block,
        n, out
    )
  return out.reshape(bsz, seq, -1)


# speedup vs baseline: 1.0406x; 1.0406x over previous
import functools

import jax
import jax.numpy as jnp
from jax import lax
from jax.experimental import pallas as pl
from jax.experimental.pallas import tpu as pltpu
from jax.experimental.pallas import tpu_sc as plsc

LN_EPS = 1e-12

_NC = 2
_NS = 16
_NW = _NC * _NS

_GROUP_SEQS = (64, 192, 320, 448)
_CHUNK = 128
_SEQS_PER_BLOCK = 16


def _sc_gather(ids3, table):
  nw, n_ch, chunk = ids3.shape
  e = table.shape[1]
  n = nw * n_ch * chunk
  rows_per_w = n_ch * chunk
  mesh = plsc.VectorSubcoreMesh(core_axis_name="c", subcore_axis_name="s")

  assert n_ch >= 2 and n_ch % 2 == 0

  @functools.partial(
      pl.kernel,
      mesh=mesh,
      out_type=jax.ShapeDtypeStruct((n // 2, e), jnp.int32),
      scratch_types=[
          pltpu.VMEM((n_ch, chunk), jnp.int32),
          pltpu.VMEM((chunk, e), jnp.float32),
          pltpu.VMEM((chunk, e), jnp.float32),
          pltpu.VMEM((chunk // 2, e), jnp.int32),
          pltpu.VMEM((chunk // 2, e), jnp.int32),
          pltpu.SemaphoreType.DMA,
          pltpu.SemaphoreType.DMA,
      ],
  )
  def gather_kernel(ids_hbm, tab_hbm, out_hbm, idx_v, rva, rvb, oba, obb,
                    sema, semb):
    wid = lax.axis_index("s") * _NC + lax.axis_index("c")
    base = wid * (rows_per_w // 2)
    pltpu.sync_copy(ids_hbm.at[wid], idx_v)
    bufs = ((rva, oba, sema), (rvb, obb, semb))

    def gstart(i, b):
      rv, _, sem = bufs[b]
      pltpu.async_copy(tab_hbm.at[idx_v.at[i]], rv, sem)

    def gwait(i, b):
      rv, _, sem = bufs[b]
      pltpu.make_async_copy(tab_hbm.at[idx_v.at[i]], rv, sem).wait()

    def to_bf16(b):
      rv, ob, _ = bufs[b]

      def rbody(r2, carry):
        for u in range(2):
          r = 2 * r2 + u
          for gcol in range(e // 16):
            sl = pl.ds(16 * gcol, 16)
            lo = lax.bitcast_convert_type(rv[2 * r, sl], jnp.int32)
            hi = lax.bitcast_convert_type(rv[2 * r + 1, sl], jnp.int32)
            lo16 = lax.shift_right_logical(lo + 0x8000, 16)
            hi16 = (hi + 0x8000) & jnp.int32(-65536)
            ob[r, sl] = lo16 | hi16
        return carry

      lax.fori_loop(0, chunk // 4, rbody, 0)

    def drain(i, b):
      _, ob, _ = bufs[b]
      pltpu.sync_copy(
          ob, out_hbm.at[pl.ds(base + i * (chunk // 2), chunk // 2)]
      )

    gstart(0, 0)
    gstart(1, 1)

    def body(k, carry):
      for b in range(2):
        i = 2 * k + b
        gwait(i, b)
        to_bf16(b)
        drain(i, b)
        gstart(i + 2, b)
      return carry

    lax.fori_loop(0, n_ch // 2 - 1, body, 0)
    for b in range(2):
      i = n_ch - 2 + b
      gwait(i, b)
      to_bf16(b)
      drain(i, b)

  return gather_kernel(ids3, table)


def _tc_fused_group(x, rows, pos_tiled, g, b, proj, bias, blk0, n_total,
                    out_buf):
  e = x.shape[1]
  r = pos_tiled.shape[0]
  h = proj.shape[1]
  grid_n = rows // r
  row0 = blk0

  def body(*refs):
    x_ref, p_ref, g_ref, b_ref, k_ref, bias_ref = refs[:6]
    o_ref = refs[-1]
    xw = pltpu.bitcast(x_ref[...], jnp.bfloat16)
    xv = xw.astype(jnp.float32) + p_ref[...]
    mu = jnp.mean(xv, axis=1, keepdims=True)
    xc = xv - mu
    var = jnp.mean(xc * xc, axis=1, keepdims=True)
    y = xc * lax.rsqrt(var + LN_EPS) * g_ref[...] + b_ref[...]
    o_ref[...] = (
        jnp.dot(y, k_ref[...], preferred_element_type=jnp.float32)
        + bias_ref[...]
    )

  in_specs = [
      pl.BlockSpec((r // 2, e), lambda i: (i, 0)),
      pl.BlockSpec((r, e), lambda i: (0, 0)),
      pl.BlockSpec((1, e), lambda i: (0, 0)),
      pl.BlockSpec((1, e), lambda i: (0, 0)),
      pl.BlockSpec((e, h), lambda i: (0, 0)),
      pl.BlockSpec((1, h), lambda i: (0, 0)),
  ]
  args = [x, pos_tiled, g, b, proj, bias]
  aliases = {}
  if out_buf is not None:
    in_specs.append(pl.BlockSpec(memory_space=pl.ANY))
    args.append(out_buf)
    aliases = {6: 0}

  return pl.pallas_call(
      body,
      grid=(grid_n,),
      in_specs=in_specs,
      out_specs=pl.BlockSpec((r, h), lambda i: (row0 + i, 0)),
      out_shape=jax.ShapeDtypeStruct((n_total, h), jnp.float32),
      input_output_aliases=aliases,
  )(*args)


def kernel(input_ids, word_emb, pos_emb, ln_scale, ln_bias, kernel, bias):
  bsz, seq = input_ids.shape
  n = bsz * seq
  ids_flat = input_ids.astype(jnp.int32).reshape(-1)

  bounds = []
  row = 0
  for gs in _GROUP_SEQS:
    bounds.append((row, gs * seq))
    row += gs * seq

  unit = 2 * _NW * _CHUNK
  gathered = []
  for r0, rows in bounds:
    rows_pad = -(-rows // unit) * unit
    ids_g = lax.dynamic_slice(ids_flat, (r0,), (rows,))
    if rows_pad != rows:
      ids_g = jnp.concatenate([ids_g, ids_g[: rows_pad - rows]])
    gathered.append(
        _sc_gather(
            ids_g.reshape(_NW, rows_pad // (_NW * _CHUNK), _CHUNK), word_emb
        )
    )

  r_block = _SEQS_PER_BLOCK * seq
  pos_tiled = jnp.tile(pos_emb[:seq], (_SEQS_PER_BLOCK, 1))
  g2 = ln_scale[None, :]
  b2 = ln_bias[None, :]
  bias2 = bias[None, :]
  out = None
  for i, (r0, rows) in enumerate(bounds):
    out = _tc_fused_group(
        gathered[i], rows, pos_tiled, g2, b2, kernel, bias2, r0 // r_block,
        n, out
    )
  return out.reshape(bsz, seq, -1)
